# baseline (speedup 1.0000x reference)
import functools

import jax
import jax.numpy as jnp
from jax.experimental import pallas as pl
import jax.experimental.pallas.tpu as pltpu

B, L, CLEN, E, TOP_K, D_MODEL = 256, 8, 128, 64, 2, 128
IN_F = 3 * CLEN
EPS = 1e-9


def _moe_body(flat_ref, logits_ref, maskf_ref, w_ref, b_ref, out_ref,
              p_sc, acc_sc):
    e = pl.program_id(0)

    @pl.when(e == 0)
    def _():
        logit = logits_ref[:]
        m = jnp.max(logit, axis=1, keepdims=True)
        ex = jnp.exp(logit - m)
        g = ex / jnp.sum(ex, axis=1, keepdims=True)
        g = g * maskf_ref[:]
        lane = jax.lax.broadcasted_iota(jnp.int32, (B, E), 1)
        v1 = jnp.max(g, axis=1, keepdims=True)
        i1 = jnp.min(jnp.where(g == v1, lane, E), axis=1, keepdims=True)
        oh1 = lane == i1
        g2 = jnp.where(oh1, -1.0, g)
        v2 = jnp.max(g2, axis=1, keepdims=True)
        i2 = jnp.min(jnp.where(g2 == v2, lane, E), axis=1, keepdims=True)
        oh2 = lane == i2
        denom = v1 + v2 + EPS
        p = (jnp.where(oh1, v1, 0.0) + jnp.where(oh2, v2, 0.0)) / denom
        p_sc[:] = p
        bias = jnp.dot(p, b_ref[:], preferred_element_type=jnp.float32)
        acc_sc[:] = jnp.broadcast_to(bias[:, None, :], (B, L, D_MODEL))

    lane = jax.lax.broadcasted_iota(jnp.int32, (B, E), 1)
    pcol = jnp.sum(jnp.where(lane == e, p_sc[:], 0.0), axis=1,
                   keepdims=True)
    y = jnp.dot(flat_ref[:], w_ref[0],
                preferred_element_type=jnp.float32)
    acc_sc[:] += pcol[:, :, None] * y.reshape(B, L, D_MODEL)

    @pl.when(e == E - 1)
    def _():
        out_ref[:] = acc_sc[:].astype(jnp.bfloat16)


@jax.jit
def kernel(cycle_curve_data, logits, moe_masks, W, b):
    flat = cycle_curve_data.reshape(B * L, IN_F).astype(jnp.bfloat16)
    maskf = (moe_masks == 1).astype(jnp.float32)
    wb = W.astype(jnp.bfloat16)

    out = pl.pallas_call(
        _moe_body,
        grid=(E,),
        in_specs=[
            pl.BlockSpec((B * L, IN_F), lambda e: (0, 0)),
            pl.BlockSpec((B, E), lambda e: (0, 0)),
            pl.BlockSpec((B, E), lambda e: (0, 0)),
            pl.BlockSpec((1, IN_F, D_MODEL), lambda e: (e, 0, 0)),
            pl.BlockSpec((E, D_MODEL), lambda e: (0, 0)),
        ],
        out_specs=pl.BlockSpec((B, L, D_MODEL), lambda e: (0, 0, 0)),
        out_shape=jax.ShapeDtypeStruct((B, L, D_MODEL), jnp.bfloat16),
        scratch_shapes=[
            pltpu.VMEM((B, E), jnp.float32),
            pltpu.VMEM((B, L, D_MODEL), jnp.float32),
        ],
    )(flat, logits, maskf, wb, b)
    return out


# speedup vs baseline: 1.4213x; 1.4213x over previous
import functools

import jax
import jax.numpy as jnp
from jax.experimental import pallas as pl
import jax.experimental.pallas.tpu as pltpu

B, L, CLEN, E, TOP_K, D_MODEL = 256, 8, 128, 64, 2, 128
IN_F = 3 * CLEN
EPS = 1e-9


EB = 8


def _moe_body(flat_ref, logits_ref, maskf_ref, w_ref, b_ref, out_ref,
              p_sc, acc_sc):
    step = pl.program_id(0)

    @pl.when(step == 0)
    def _():
        logit = logits_ref[:]
        m = jnp.max(logit, axis=1, keepdims=True)
        ex = jnp.exp(logit - m)
        g = ex / jnp.sum(ex, axis=1, keepdims=True)
        g = g * maskf_ref[:]
        lane = jax.lax.broadcasted_iota(jnp.int32, (B, E), 1)
        v1 = jnp.max(g, axis=1, keepdims=True)
        i1 = jnp.min(jnp.where(g == v1, lane, E), axis=1, keepdims=True)
        oh1 = lane == i1
        g2 = jnp.where(oh1, -1.0, g)
        v2 = jnp.max(g2, axis=1, keepdims=True)
        i2 = jnp.min(jnp.where(g2 == v2, lane, E), axis=1, keepdims=True)
        oh2 = lane == i2
        denom = v1 + v2 + EPS
        p = (jnp.where(oh1, v1, 0.0) + jnp.where(oh2, v2, 0.0)) / denom
        p_sc[:] = p
        bias = jnp.dot(p, b_ref[:], preferred_element_type=jnp.float32)
        acc_sc[:] = jnp.broadcast_to(bias[:, None, :], (B, L, D_MODEL))

    lane = jax.lax.broadcasted_iota(jnp.int32, (B, E), 1)
    p = p_sc[:]
    flat = flat_ref[:]
    upd = jnp.zeros((B, L, D_MODEL), jnp.float32)
    for j in range(EB):
        e = step * EB + j
        pcol = jnp.sum(jnp.where(lane == e, p, 0.0), axis=1,
                       keepdims=True)
        y = jnp.dot(flat, w_ref[j],
                    preferred_element_type=jnp.float32)
        upd = upd + pcol[:, :, None] * y.reshape(B, L, D_MODEL)
    acc_sc[:] += upd

    @pl.when(step == (E // EB) - 1)
    def _():
        out_ref[:] = acc_sc[:].astype(jnp.bfloat16)


@jax.jit
def kernel(cycle_curve_data, logits, moe_masks, W, b):
    flat = cycle_curve_data.reshape(B * L, IN_F).astype(jnp.bfloat16)
    maskf = (moe_masks == 1).astype(jnp.float32)
    wb = W.astype(jnp.bfloat16)

    out = pl.pallas_call(
        _moe_body,
        grid=(E // EB,),
        in_specs=[
            pl.BlockSpec((B * L, IN_F), lambda e: (0, 0)),
            pl.BlockSpec((B, E), lambda e: (0, 0)),
            pl.BlockSpec((B, E), lambda e: (0, 0)),
            pl.BlockSpec((EB, IN_F, D_MODEL), lambda e: (e, 0, 0)),
            pl.BlockSpec((E, D_MODEL), lambda e: (0, 0)),
        ],
        out_specs=pl.BlockSpec((B, L, D_MODEL), lambda e: (0, 0, 0)),
        out_shape=jax.ShapeDtypeStruct((B, L, D_MODEL), jnp.bfloat16),
        scratch_shapes=[
            pltpu.VMEM((B, E), jnp.float32),
            pltpu.VMEM((B, L, D_MODEL), jnp.float32),
        ],
    )(flat, logits, maskf, wb, b)
    return out


# speedup vs baseline: 1.8146x; 1.2767x over previous
import functools

import jax
import jax.numpy as jnp
from jax.experimental import pallas as pl
import jax.experimental.pallas.tpu as pltpu

B, L, CLEN, E, TOP_K, D_MODEL = 256, 8, 128, 64, 2, 128
IN_F = 3 * CLEN
EPS = 1e-9


EB = 8


def _moe_body(flat_ref, logits_ref, maskf_ref, w_ref, b_ref, out_ref,
              p_sc, flatb_sc, acc_sc):
    step = pl.program_id(0)

    @pl.when(step == 0)
    def _():
        logit = logits_ref[:]
        m = jnp.max(logit, axis=1, keepdims=True)
        ex = jnp.exp(logit - m)
        g = ex / jnp.sum(ex, axis=1, keepdims=True)
        g = g * maskf_ref[:]
        lane = jax.lax.broadcasted_iota(jnp.int32, (B, E), 1)
        v1 = jnp.max(g, axis=1, keepdims=True)
        i1 = jnp.min(jnp.where(g == v1, lane, E), axis=1, keepdims=True)
        oh1 = lane == i1
        g2 = jnp.where(oh1, -1.0, g)
        v2 = jnp.max(g2, axis=1, keepdims=True)
        i2 = jnp.min(jnp.where(g2 == v2, lane, E), axis=1, keepdims=True)
        oh2 = lane == i2
        denom = v1 + v2 + EPS
        p = (jnp.where(oh1, v1, 0.0) + jnp.where(oh2, v2, 0.0)) / denom
        p_sc[:] = p
        bias = jnp.dot(p, b_ref[:], preferred_element_type=jnp.float32)
        acc_sc[:] = jnp.broadcast_to(bias[:, None, :], (B, L, D_MODEL))
        flatb_sc[:] = flat_ref[:].astype(jnp.bfloat16)

    lane = jax.lax.broadcasted_iota(jnp.int32, (B, E), 1)
    p = p_sc[:]
    flat = flatb_sc[:]
    upd = jnp.zeros((B, L, D_MODEL), jnp.float32)
    for j in range(EB):
        e = step * EB + j
        pcol = jnp.sum(jnp.where(lane == e, p, 0.0), axis=1,
                       keepdims=True)
        y = jnp.dot(flat, w_ref[j].astype(jnp.bfloat16),
                    preferred_element_type=jnp.float32)
        upd = upd + pcol[:, :, None] * y.reshape(B, L, D_MODEL)
    acc_sc[:] += upd

    @pl.when(step == (E // EB) - 1)
    def _():
        out_ref[:] = acc_sc[:].astype(jnp.bfloat16)


@jax.jit
def kernel(cycle_curve_data, logits, moe_masks, W, b):
    flat = cycle_curve_data.reshape(B * L, IN_F)
    maskf = (moe_masks == 1).astype(jnp.float32)

    out = pl.pallas_call(
        _moe_body,
        grid=(E // EB,),
        in_specs=[
            pl.BlockSpec((B * L, IN_F), lambda e: (0, 0)),
            pl.BlockSpec((B, E), lambda e: (0, 0)),
            pl.BlockSpec((B, E), lambda e: (0, 0)),
            pl.BlockSpec((EB, IN_F, D_MODEL), lambda e: (e, 0, 0)),
            pl.BlockSpec((E, D_MODEL), lambda e: (0, 0)),
        ],
        out_specs=pl.BlockSpec((B, L, D_MODEL), lambda e: (0, 0, 0)),
        out_shape=jax.ShapeDtypeStruct((B, L, D_MODEL), jnp.bfloat16),
        scratch_shapes=[
            pltpu.VMEM((B, E), jnp.float32),
            pltpu.VMEM((B * L, IN_F), jnp.bfloat16),
            pltpu.VMEM((B, L, D_MODEL), jnp.float32),
        ],
    )(flat, logits, maskf, W, b)
    return out


# speedup vs baseline: 2.2530x; 1.2416x over previous
import functools

import jax
import jax.numpy as jnp
from jax.experimental import pallas as pl
import jax.experimental.pallas.tpu as pltpu

B, L, CLEN, E, TOP_K, D_MODEL = 256, 8, 128, 64, 2, 128
IN_F = 3 * CLEN
EPS = 1e-9


EB = 8


def _moe_body(flat_ref, logits_ref, maskf_ref, w_ref, b_ref, out_ref,
              p_sc, flatb_sc, acc_sc):
    step = pl.program_id(0)

    @pl.when(step == 0)
    def _():
        logit = logits_ref[:]
        m = jnp.max(logit, axis=1, keepdims=True)
        ex = jnp.exp(logit - m)
        g = ex / jnp.sum(ex, axis=1, keepdims=True)
        g = g * maskf_ref[:]
        lane = jax.lax.broadcasted_iota(jnp.int32, (B, E), 1)
        v1 = jnp.max(g, axis=1, keepdims=True)
        i1 = jnp.min(jnp.where(g == v1, lane, E), axis=1, keepdims=True)
        oh1 = lane == i1
        g2 = jnp.where(oh1, -1.0, g)
        v2 = jnp.max(g2, axis=1, keepdims=True)
        i2 = jnp.min(jnp.where(g2 == v2, lane, E), axis=1, keepdims=True)
        oh2 = lane == i2
        denom = v1 + v2 + EPS
        p = (jnp.where(oh1, v1, 0.0) + jnp.where(oh2, v2, 0.0)) / denom
        p_sc[:] = p
        bias = jnp.dot(p, b_ref[:], preferred_element_type=jnp.float32)
        acc_sc[:] = jnp.broadcast_to(bias[:, None, :], (B, L, D_MODEL))
        flatb_sc[:] = flat_ref[:].astype(jnp.bfloat16)

    lane = jax.lax.broadcasted_iota(jnp.int32, (B, E), 1)
    p = p_sc[:]
    flat = flatb_sc[:]
    wcat = jnp.concatenate(
        [w_ref[j].astype(jnp.bfloat16) for j in range(EB)], axis=1)
    yall = jnp.dot(flat, wcat,
                   preferred_element_type=jnp.float32)
    upd = jnp.zeros((B, L, D_MODEL), jnp.float32)
    for j in range(EB):
        e = step * EB + j
        pcol = jnp.sum(jnp.where(lane == e, p, 0.0), axis=1,
                       keepdims=True)
        y = yall[:, j * D_MODEL:(j + 1) * D_MODEL]
        upd = upd + pcol[:, :, None] * y.reshape(B, L, D_MODEL)
    acc_sc[:] += upd

    @pl.when(step == (E // EB) - 1)
    def _():
        out_ref[:] = acc_sc[:].astype(jnp.bfloat16)


@jax.jit
def kernel(cycle_curve_data, logits, moe_masks, W, b):
    flat = cycle_curve_data.reshape(B * L, IN_F)
    maskf = (moe_masks == 1).astype(jnp.float32)

    out = pl.pallas_call(
        _moe_body,
        grid=(E // EB,),
        in_specs=[
            pl.BlockSpec((B * L, IN_F), lambda e: (0, 0)),
            pl.BlockSpec((B, E), lambda e: (0, 0)),
            pl.BlockSpec((B, E), lambda e: (0, 0)),
            pl.BlockSpec((EB, IN_F, D_MODEL), lambda e: (e, 0, 0)),
            pl.BlockSpec((E, D_MODEL), lambda e: (0, 0)),
        ],
        out_specs=pl.BlockSpec((B, L, D_MODEL), lambda e: (0, 0, 0)),
        out_shape=jax.ShapeDtypeStruct((B, L, D_MODEL), jnp.bfloat16),
        scratch_shapes=[
            pltpu.VMEM((B, E), jnp.float32),
            pltpu.VMEM((B * L, IN_F), jnp.bfloat16),
            pltpu.VMEM((B, L, D_MODEL), jnp.float32),
        ],
    )(flat, logits, maskf, W, b)
    return out
